# baseline (speedup 1.0000x reference)
import functools

import jax
import jax.numpy as jnp
from jax.experimental import pallas as pl
from jax.experimental.pallas import tpu as pltpu
from jax.experimental.pallas import tpu_sc as plsc

EMBED_DIM = 32
CHUNK = 128
CHUNKS_PER_STEP = 8
IDX_PER_STEP = CHUNK * CHUNKS_PER_STEP
TBLOCK = 8192


def _transpose_table(t_t):
    V = t_t.shape[1]

    def body(in_ref, out_ref):
        out_ref[...] = in_ref[...].T

    return pl.pallas_call(
        body,
        grid=(pl.cdiv(V, TBLOCK),),
        in_specs=[pl.BlockSpec((EMBED_DIM, TBLOCK), lambda i: (0, i))],
        out_specs=pl.BlockSpec((TBLOCK, EMBED_DIM), lambda i: (i, 0)),
        out_shape=jax.ShapeDtypeStruct((V, EMBED_DIM), jnp.float32),
        compiler_params=pltpu.CompilerParams(
            dimension_semantics=("parallel",)
        ),
    )(t_t)


@jax.jit
def kernel(x, table):
    B, L = x.shape
    N = B * L
    idx = x.reshape(N // CHUNK, CHUNK)
    table_rm = _transpose_table(jnp.transpose(table))
    mesh = plsc.VectorSubcoreMesh(core_axis_name="c", subcore_axis_name="s")

    @functools.partial(
        pl.kernel,
        out_type=jax.ShapeDtypeStruct((N, EMBED_DIM), table.dtype),
        mesh=mesh,
        compiler_params=pltpu.CompilerParams(use_tc_tiling_on_sc=False),
        scratch_types=[pltpu.SemaphoreType.DMA],
    )
    def gather_kernel(table_hbm, idx_hbm, out_hbm, sem):
        def body(idx_vmem, out_vmem):
            copies = [
                pltpu.async_copy(
                    table_hbm.at[idx_vmem.at[j]],
                    out_vmem.at[pl.ds(j * CHUNK, CHUNK)],
                    sem,
                )
                for j in range(CHUNKS_PER_STEP)
            ]
            for c in copies:
                c.wait()

        pltpu.emit_pipeline(
            body,
            grid=(N // IDX_PER_STEP,),
            in_specs=[
                pl.BlockSpec((CHUNKS_PER_STEP, CHUNK), lambda i: (i, 0))
            ],
            out_specs=[
                pl.BlockSpec((IDX_PER_STEP, EMBED_DIM), lambda i: (i, 0))
            ],
            core_axis_name=("c", "s"),
            dimension_semantics=(pltpu.PARALLEL,),
        )(idx_hbm, out_hbm)

    return gather_kernel(table_rm, idx).reshape(B, L, EMBED_DIM)


# speedup vs baseline: 1.3540x; 1.3540x over previous
import functools

import jax
import jax.numpy as jnp
from jax.experimental import pallas as pl
from jax.experimental.pallas import tpu as pltpu
from jax.experimental.pallas import tpu_sc as plsc

EMBED_DIM = 32
CHUNK = 128
CHUNKS_PER_STEP = 8
IDX_PER_STEP = CHUNK * CHUNKS_PER_STEP
TBLOCK = 8192
OBLOCK = 4096


def _transpose_table(t_t):
    V = t_t.shape[1]

    def body(in_ref, out_ref):
        s = in_ref[...].T.reshape(TBLOCK // 4, 4, EMBED_DIM)
        out_ref[...] = jnp.concatenate(
            [s[:, k, :] for k in range(4)], axis=1
        )

    return pl.pallas_call(
        body,
        grid=(pl.cdiv(V, TBLOCK),),
        in_specs=[pl.BlockSpec((EMBED_DIM, TBLOCK), lambda i: (0, i))],
        out_specs=pl.BlockSpec((TBLOCK // 4, 128), lambda i: (i, 0)),
        out_shape=jax.ShapeDtypeStruct((V // 4, 128), jnp.float32),
        compiler_params=pltpu.CompilerParams(
            dimension_semantics=("parallel",)
        ),
    )(t_t)


def _transpose_out(g_flat, L, B):
    def body(in_ref, out_ref):
        v = in_ref[...]
        parts = [
            v[:, 32 * k:32 * (k + 1)] for k in range(4)
        ]
        stacked = jnp.stack(parts, axis=1).reshape(OBLOCK, EMBED_DIM)
        out_ref[...] = stacked.T

    blocks_per_l = B // OBLOCK
    in_rows = OBLOCK * EMBED_DIM // 128

    return pl.pallas_call(
        body,
        grid=(L, blocks_per_l),
        in_specs=[
            pl.BlockSpec(
                (in_rows, 128),
                lambda l, i: (l * blocks_per_l + i, 0),
            )
        ],
        out_specs=pl.BlockSpec(
            (EMBED_DIM, OBLOCK), lambda l, i: (l, i)
        ),
        out_shape=jax.ShapeDtypeStruct((L * EMBED_DIM, B), jnp.float32),
        compiler_params=pltpu.CompilerParams(
            dimension_semantics=("parallel", "parallel")
        ),
    )(g_flat)


@jax.jit
def kernel(x, table):
    B, L = x.shape
    V = table.shape[0]
    N = B * L
    x_t = jnp.transpose(x)
    table_rm = _transpose_table(jnp.transpose(table)).reshape(V, EMBED_DIM)
    mesh = plsc.VectorSubcoreMesh(core_axis_name="c", subcore_axis_name="s")
    steps_per_l = B // IDX_PER_STEP

    @functools.partial(
        pl.kernel,
        out_type=jax.ShapeDtypeStruct((N, EMBED_DIM), table.dtype),
        mesh=mesh,
        compiler_params=pltpu.CompilerParams(use_tc_tiling_on_sc=False),
        scratch_types=[pltpu.SemaphoreType.DMA],
    )
    def gather_kernel(table_hbm, idx_hbm, out_hbm, sem):
        def body(idx_vmem, out_vmem):
            copies = [
                pltpu.async_copy(
                    table_hbm.at[idx_vmem.at[0, pl.ds(j * CHUNK, CHUNK)]],
                    out_vmem.at[pl.ds(j * CHUNK, CHUNK)],
                    sem,
                )
                for j in range(CHUNKS_PER_STEP)
            ]
            for c in copies:
                c.wait()

        pltpu.emit_pipeline(
            body,
            grid=(L, steps_per_l),
            in_specs=[
                pl.BlockSpec((1, IDX_PER_STEP), lambda l, i: (l, i))
            ],
            out_specs=[
                pl.BlockSpec(
                    (IDX_PER_STEP, EMBED_DIM),
                    lambda l, i: (l * steps_per_l + i, 0),
                )
            ],
            core_axis_name=("c", "s"),
            dimension_semantics=(pltpu.PARALLEL, pltpu.PARALLEL),
        )(idx_hbm, out_hbm)

    g = gather_kernel(table_rm, x_t)
    p = _transpose_out(g.reshape(N * EMBED_DIM // 128, 128), L, B)
    return p.reshape(L, EMBED_DIM, B).transpose(2, 0, 1)


# speedup vs baseline: 2.0178x; 1.4902x over previous
import functools

import jax
import jax.numpy as jnp
from jax.experimental import pallas as pl
from jax.experimental.pallas import tpu as pltpu
from jax.experimental.pallas import tpu_sc as plsc

EMBED_DIM = 32
CHUNK = 128
CHUNKS_PER_STEP = 8
IDX_PER_STEP = CHUNK * CHUNKS_PER_STEP
OUT_MINOR = CHUNK * EMBED_DIM
TBLOCK = 8192


def _transpose_table(t_t):
    V = t_t.shape[1]

    def body(in_ref, out_ref):
        s = in_ref[...].T.reshape(TBLOCK // 4, 4, EMBED_DIM)
        out_ref[...] = jnp.concatenate(
            [s[:, k, :] for k in range(4)], axis=1
        )

    return pl.pallas_call(
        body,
        grid=(pl.cdiv(V, TBLOCK),),
        in_specs=[pl.BlockSpec((EMBED_DIM, TBLOCK), lambda i: (0, i))],
        out_specs=pl.BlockSpec((TBLOCK // 4, 128), lambda i: (i, 0)),
        out_shape=jax.ShapeDtypeStruct((V // 4, 128), jnp.float32),
        compiler_params=pltpu.CompilerParams(
            dimension_semantics=("parallel",)
        ),
    )(t_t)


@jax.jit
def kernel(x, table):
    B, L = x.shape
    V = table.shape[0]
    N = B * L
    x_t = jnp.transpose(x)
    table_rm = _transpose_table(jnp.transpose(table)).reshape(V, EMBED_DIM)
    mesh = plsc.VectorSubcoreMesh(core_axis_name="c", subcore_axis_name="s")
    steps_per_l = B // IDX_PER_STEP
    rows_per_step = IDX_PER_STEP * EMBED_DIM // OUT_MINOR

    @functools.partial(
        pl.kernel,
        out_type=jax.ShapeDtypeStruct((N, EMBED_DIM), table.dtype),
        mesh=mesh,
        compiler_params=pltpu.CompilerParams(use_tc_tiling_on_sc=False),
        scratch_types=[pltpu.SemaphoreType.DMA],
    )
    def gather_kernel(table_hbm, idx_hbm, out_hbm, sem):
        def body(idx_vmem, out_vmem):
            copies = [
                pltpu.async_copy(
                    table_hbm.at[idx_vmem.at[0, pl.ds(j * CHUNK, CHUNK)]],
                    out_vmem.at[pl.ds(j * CHUNK, CHUNK)],
                    sem,
                )
                for j in range(CHUNKS_PER_STEP)
            ]
            for c in copies:
                c.wait()

        pltpu.emit_pipeline(
            body,
            grid=(L, steps_per_l),
            in_specs=[
                pl.BlockSpec((1, IDX_PER_STEP), lambda l, i: (l, i))
            ],
            out_specs=[
                pl.BlockSpec(
                    (IDX_PER_STEP, EMBED_DIM),
                    lambda l, i: (l * steps_per_l + i, 0),
                )
            ],
            core_axis_name=("c", "s"),
            dimension_semantics=(pltpu.PARALLEL, pltpu.PARALLEL),
        )(idx_hbm, out_hbm)

    g = gather_kernel(table_rm, x_t)
    return g.reshape(L, B, EMBED_DIM).transpose(1, 0, 2)
